# baseline (speedup 1.0000x reference)
import functools

import jax
import jax.numpy as jnp
from jax import lax
from jax.experimental import pallas as pl
from jax.experimental.pallas import tpu as pltpu
from jax.experimental.pallas import tpu_sc as plsc

_N = 50000
_K = 10
_V = 100000
_D = 128

_NC, _NS = 2, 16
_C = 32
_TOTC = 1600
_CPS = _TOTC // _NS
_NC0, _NC1 = 50, 50
_NI = _C * _K
_STREAMS = ((0, 128), (128, 128), (256, 64))

_mesh = plsc.VectorSubcoreMesh(
    core_axis_name="c", subcore_axis_name="s", num_cores=_NC, num_subcores=_NS
)


@functools.partial(
    pl.kernel,
    out_type=jax.ShapeDtypeStruct((_N, _D), jnp.float32),
    mesh=_mesh,
    scratch_types=[
        pltpu.VMEM((_NI,), jnp.int32),
        pltpu.VMEM((_NI,), jnp.int32),
        pltpu.VMEM((_C,), jnp.int32),
        pltpu.VMEM((_C,), jnp.int32),
        pltpu.VMEM((_NI + 16,), jnp.float32),
        pltpu.VMEM((_NI + 16,), jnp.float32),
        pltpu.VMEM((_NI, _D), jnp.float32),
        pltpu.VMEM((_NI, _D), jnp.float32),
        pltpu.VMEM((_C, _D), jnp.float32),
        pltpu.VMEM((_C, _D), jnp.float32),
        pltpu.VMEM((_C, _D), jnp.float32),
        pltpu.VMEM((_C, _D), jnp.float32),
        pltpu.SemaphoreType.DMA,
        pltpu.SemaphoreType.DMA,
        pltpu.SemaphoreType.DMA,
        pltpu.SemaphoreType.DMA,
        pltpu.SemaphoreType.DMA,
        pltpu.SemaphoreType.DMA,
    ],
)
def _sc_aggregate(meta_i_hbm, pocket_hbm, meta_f_hbm, table_hbm, out_hbm,
                  nidx0, nidx1, self0, self1, wm0, wm1,
                  nrows0, nrows1, srows0, srows1, outv0, outv1,
                  sm0, sm1, sr0, sr1, so0, so1):
    core = lax.axis_index("c")
    sub = lax.axis_index("s")
    chunk0 = sub * _CPS + jnp.where(core == 0, 0, _NC0)
    n_my = jnp.where(core == 0, _NC0, _NC1)

    def node_base(g_local):
        return jnp.minimum((chunk0 + g_local) * _C, _N - _C)

    def meta_descs(g_local, iv, sv, wv, sem):
        nb = node_base(g_local)
        return [
            pltpu.make_async_copy(meta_i_hbm.at[pl.ds(nb * _K, _NI)], iv,
                                  sem),
            pltpu.make_async_copy(pocket_hbm.at[pl.ds(nb, _C)], sv, sem),
            pltpu.make_async_copy(meta_f_hbm.at[pl.ds(_N * _K + nb * _K, _NI)],
                                  wv.at[pl.ds(0, _NI)], sem),
        ]

    def gather_descs(iv, sv, nr, srws, sem):
        descs = [
            pltpu.make_async_copy(
                table_hbm.at[iv.at[pl.ds(off, cnt)]],
                nr.at[pl.ds(off, cnt), :], sem)
            for off, cnt in _STREAMS
        ]
        descs.append(pltpu.make_async_copy(table_hbm.at[sv], srws, sem))
        return descs

    def store_desc(g_local, ov, sem):
        return pltpu.make_async_copy(
            ov, out_hbm.at[pl.ds(node_base(g_local), _C), :], sem)

    lanes = lax.iota(jnp.int32, 16)
    valid = lanes < _K

    def compute(wv, nr, srws, ov):
        def node_body(i, carry):
            wmv = jnp.where(valid, wv[pl.ds(i * _K, 16)], 0.0)
            wks = [wmv[k] for k in range(_K)]
            denom = 1.0 + wks[0]
            for k in range(1, _K):
                denom = denom + wks[k]
            inv = 1.0 / jnp.maximum(jnp.full((16,), denom, jnp.float32), 1e-6)
            rbase = i * _K
            acc = [None] * (_D // 16)
            for d in range(_D // 16):
                acc[d] = srws[i, pl.ds(d * 16, 16)]
            for k in range(_K):
                wk = wks[k]
                for d in range(_D // 16):
                    acc[d] = acc[d] + wk * nr[rbase + k, pl.ds(d * 16, 16)]
            for d in range(_D // 16):
                ov[i, pl.ds(d * 16, 16)] = acc[d] * inv
            return carry

        lax.fori_loop(0, _C, node_body, 0)

    for d in meta_descs(0, nidx0, self0, wm0, sm0):
        d.start()
    for d in meta_descs(1, nidx1, self1, wm1, sm1):
        d.start()
    for d in meta_descs(0, nidx0, self0, wm0, sm0):
        d.wait()
    for d in gather_descs(nidx0, self0, nrows0, srows0, sr0):
        d.start()

    def chunk_pair(i, carry):
        cA = 2 * i
        cB = 2 * i + 1
        for d in meta_descs(lax.rem(cA + 1, n_my), nidx1, self1, wm1, sm1):
            d.wait()
        for d in gather_descs(nidx1, self1, nrows1, srows1, sr1):
            d.start()
        for d in gather_descs(nidx0, self0, nrows0, srows0, sr0):
            d.wait()

        @pl.when(cA >= 2)
        def _():
            store_desc(cA - 2, outv0, so0).wait()

        compute(wm0, nrows0, srows0, outv0)
        store_desc(cA, outv0, so0).start()
        for d in meta_descs(lax.rem(cA + 2, n_my), nidx0, self0, wm0, sm0):
            d.start()
        for d in meta_descs(lax.rem(cB + 1, n_my), nidx0, self0, wm0, sm0):
            d.wait()
        for d in gather_descs(nidx0, self0, nrows0, srows0, sr0):
            d.start()
        for d in gather_descs(nidx1, self1, nrows1, srows1, sr1):
            d.wait()

        @pl.when(cB >= 2)
        def _():
            store_desc(cB - 2, outv1, so1).wait()

        compute(wm1, nrows1, srows1, outv1)
        store_desc(cB, outv1, so1).start()
        for d in meta_descs(lax.rem(cB + 2, n_my), nidx1, self1, wm1, sm1):
            d.start()
        return carry

    lax.fori_loop(0, n_my // 2, chunk_pair, 0)

    for d in meta_descs(1, nidx1, self1, wm1, sm1):
        d.wait()
    for d in gather_descs(nidx0, self0, nrows0, srows0, sr0):
        d.wait()
    store_desc(n_my - 2, outv0, so0).wait()
    store_desc(n_my - 1, outv1, so1).wait()


def kernel(embed_weight, neighbor_weight, neighbor_mask, nodes_pocket, neighbor_idx):
    wm_bits = lax.bitcast_convert_type(neighbor_weight * neighbor_mask,
                                       jnp.int32)
    meta_i = jnp.stack(
        [neighbor_idx.astype(jnp.int32), wm_bits]).reshape(-1)
    meta_f = lax.bitcast_convert_type(meta_i, jnp.float32)
    return _sc_aggregate(meta_i, nodes_pocket.astype(jnp.int32), meta_f,
                         embed_weight)


# speedup vs baseline: 1.0253x; 1.0253x over previous
import functools

import jax
import jax.numpy as jnp
from jax import lax
from jax.experimental import pallas as pl
from jax.experimental.pallas import tpu as pltpu
from jax.experimental.pallas import tpu_sc as plsc

_N = 50000
_K = 10
_V = 100000
_D = 128

_NC, _NS = 2, 16
_C = 32
_TOTC = 1600
_CPS = _TOTC // _NS
_NC0, _NC1 = 50, 50
_NI = _C * _K
_STREAMS = ((0, 128), (128, 128), (256, 64))

_mesh = plsc.VectorSubcoreMesh(
    core_axis_name="c", subcore_axis_name="s", num_cores=_NC, num_subcores=_NS
)


@functools.partial(
    pl.kernel,
    out_type=jax.ShapeDtypeStruct((_N, _D), jnp.float32),
    mesh=_mesh,
    scratch_types=[
        pltpu.VMEM((_NI,), jnp.int32),
        pltpu.VMEM((_NI,), jnp.int32),
        pltpu.VMEM((_C,), jnp.int32),
        pltpu.VMEM((_C,), jnp.int32),
        pltpu.VMEM((_NI + 16,), jnp.float32),
        pltpu.VMEM((_NI + 16,), jnp.float32),
        pltpu.VMEM((_NI, _D), jnp.float32),
        pltpu.VMEM((_NI, _D), jnp.float32),
        pltpu.VMEM((_C, _D), jnp.float32),
        pltpu.VMEM((_C, _D), jnp.float32),
        pltpu.VMEM((_C, _D), jnp.float32),
        pltpu.VMEM((_C, _D), jnp.float32),
        pltpu.SemaphoreType.DMA,
        pltpu.SemaphoreType.DMA,
        pltpu.SemaphoreType.DMA,
        pltpu.SemaphoreType.DMA,
        pltpu.SemaphoreType.DMA,
        pltpu.SemaphoreType.DMA,
    ],
)
def _sc_aggregate(nidx_hbm, pocket_hbm, wm_hbm, table_hbm, out_hbm,
                  nidx0, nidx1, self0, self1, wm0, wm1,
                  nrows0, nrows1, srows0, srows1, outv0, outv1,
                  sm0, sm1, sr0, sr1, so0, so1):
    core = lax.axis_index("c")
    sub = lax.axis_index("s")
    chunk0 = sub * _CPS + jnp.where(core == 0, 0, _NC0)
    n_my = jnp.where(core == 0, _NC0, _NC1)

    def node_base(g_local):
        return jnp.minimum((chunk0 + g_local) * _C, _N - _C)

    def meta_descs(g_local, iv, sv, wv, sem):
        nb = node_base(g_local)
        return [
            pltpu.make_async_copy(nidx_hbm.at[pl.ds(nb * _K, _NI)], iv, sem),
            pltpu.make_async_copy(pocket_hbm.at[pl.ds(nb, _C)], sv, sem),
            pltpu.make_async_copy(wm_hbm.at[pl.ds(nb * _K, _NI)],
                                  wv.at[pl.ds(0, _NI)], sem),
        ]

    def gather_descs(iv, sv, nr, srws, sem):
        descs = [
            pltpu.make_async_copy(
                table_hbm.at[iv.at[pl.ds(off, cnt)]],
                nr.at[pl.ds(off, cnt), :], sem)
            for off, cnt in _STREAMS
        ]
        descs.append(pltpu.make_async_copy(table_hbm.at[sv], srws, sem))
        return descs

    def store_desc(g_local, ov, sem):
        return pltpu.make_async_copy(
            ov, out_hbm.at[pl.ds(node_base(g_local), _C), :], sem)

    lanes = lax.iota(jnp.int32, 16)
    valid = lanes < _K

    def compute(wv, nr, srws, ov):
        def node_body(i, carry):
            wmv = jnp.where(valid, wv[pl.ds(i * _K, 16)], 0.0)
            wks = [wmv[k] for k in range(_K)]
            denom = 1.0 + wks[0]
            for k in range(1, _K):
                denom = denom + wks[k]
            inv = 1.0 / jnp.maximum(jnp.full((16,), denom, jnp.float32), 1e-6)
            rbase = i * _K
            acc = [None] * (_D // 16)
            for d in range(_D // 16):
                acc[d] = srws[i, pl.ds(d * 16, 16)]
            for k in range(_K):
                wk = wks[k]
                for d in range(_D // 16):
                    acc[d] = acc[d] + wk * nr[rbase + k, pl.ds(d * 16, 16)]
            for d in range(_D // 16):
                ov[i, pl.ds(d * 16, 16)] = acc[d] * inv
            return carry

        lax.fori_loop(0, _C, node_body, 0)

    for d in meta_descs(0, nidx0, self0, wm0, sm0):
        d.start()
    for d in meta_descs(1, nidx1, self1, wm1, sm1):
        d.start()
    for d in meta_descs(0, nidx0, self0, wm0, sm0):
        d.wait()
    for d in gather_descs(nidx0, self0, nrows0, srows0, sr0):
        d.start()

    def chunk_pair(i, carry):
        cA = 2 * i
        cB = 2 * i + 1
        for d in meta_descs(lax.rem(cA + 1, n_my), nidx1, self1, wm1, sm1):
            d.wait()
        for d in gather_descs(nidx1, self1, nrows1, srows1, sr1):
            d.start()
        for d in gather_descs(nidx0, self0, nrows0, srows0, sr0):
            d.wait()

        @pl.when(cA >= 2)
        def _():
            store_desc(cA - 2, outv0, so0).wait()

        compute(wm0, nrows0, srows0, outv0)
        store_desc(cA, outv0, so0).start()
        for d in meta_descs(lax.rem(cA + 2, n_my), nidx0, self0, wm0, sm0):
            d.start()
        for d in meta_descs(lax.rem(cB + 1, n_my), nidx0, self0, wm0, sm0):
            d.wait()
        for d in gather_descs(nidx0, self0, nrows0, srows0, sr0):
            d.start()
        for d in gather_descs(nidx1, self1, nrows1, srows1, sr1):
            d.wait()

        @pl.when(cB >= 2)
        def _():
            store_desc(cB - 2, outv1, so1).wait()

        compute(wm1, nrows1, srows1, outv1)
        store_desc(cB, outv1, so1).start()
        for d in meta_descs(lax.rem(cB + 2, n_my), nidx1, self1, wm1, sm1):
            d.start()
        return carry

    lax.fori_loop(0, n_my // 2, chunk_pair, 0)

    for d in meta_descs(1, nidx1, self1, wm1, sm1):
        d.wait()
    for d in gather_descs(nidx0, self0, nrows0, srows0, sr0):
        d.wait()
    store_desc(n_my - 2, outv0, so0).wait()
    store_desc(n_my - 1, outv1, so1).wait()


def kernel(embed_weight, neighbor_weight, neighbor_mask, nodes_pocket, neighbor_idx):
    nidx = neighbor_idx.astype(jnp.int32).reshape(-1)
    wm = (neighbor_weight * neighbor_mask).reshape(-1)
    return _sc_aggregate(nidx, nodes_pocket.astype(jnp.int32), wm,
                         embed_weight)
